# baseline (speedup 1.0000x reference)
import functools

import jax
import jax.numpy as jnp
from jax import lax
from jax.experimental import pallas as pl
from jax.experimental.pallas import tpu as pltpu
from jax.experimental.pallas import tpu_sc as plsc

LANES = 16


def _sc_kernel_body(num_chunks, chunk, d, uid_hbm, iid_hbm, ut_hbm, it_hbm,
                    out_hbm, idx_u, idx_i, ubuf, ibuf, dots, outv,
                    sem_u0, sem_u1, sem_i0, sem_i1):
    nc = 2
    wid = lax.axis_index("s") * nc + lax.axis_index("c")
    per_w = num_chunks * chunk
    base = wid * per_w
    d_steps = d // LANES

    pltpu.sync_copy(uid_hbm.at[pl.ds(base, per_w)], idx_u)
    pltpu.sync_copy(iid_hbm.at[pl.ds(base, per_w)], idx_i)

    def issue(j, slot, su, si):
        pltpu.async_copy(ut_hbm.at[idx_u.at[pl.ds(j * chunk, chunk)]],
                         ubuf.at[slot], su)
        pltpu.async_copy(it_hbm.at[idx_i.at[pl.ds(j * chunk, chunk)]],
                         ibuf.at[slot], si)

    def wait(slot, su, si):
        pltpu.make_async_copy(ut_hbm.at[idx_u.at[pl.ds(0, chunk)]],
                              ubuf.at[slot], su).wait()
        pltpu.make_async_copy(it_hbm.at[idx_i.at[pl.ds(0, chunk)]],
                              ibuf.at[slot], si).wait()

    sems = ((sem_u0, sem_i0), (sem_u1, sem_i1))
    lane = lax.iota(jnp.int32, LANES)

    issue(0, 0, sem_u0, sem_i0)
    for j in range(num_chunks):
        s = j % 2
        if j + 1 < num_chunks:
            issue(j + 1, 1 - s, *sems[1 - s])
        wait(s, *sems[s])

        @plsc.parallel_loop(0, chunk, unroll=4)
        def pair_body(p, s=s):
            m = [ubuf[s, p, pl.ds(k * LANES, LANES)] *
                 ibuf[s, p, pl.ds(k * LANES, LANES)] for k in range(d_steps)]
            while len(m) > 1:
                m = [m[2 * t] + m[2 * t + 1] for t in range(len(m) // 2)]
            dots[p, pl.ds(0, LANES)] = m[0]

        @plsc.parallel_loop(0, chunk // LANES, unroll=2)
        def group_body(g, j=j):
            row = g * LANES + lane
            acc = [plsc.load_gather(dots,
                                    [row, jnp.full((LANES,), c, jnp.int32)])
                   for c in range(LANES)]
            while len(acc) > 1:
                acc = [acc[2 * t] + acc[2 * t + 1]
                       for t in range(len(acc) // 2)]
            rating = 1.0 + 4.0 / (1.0 + jnp.exp(-acc[0]))
            outv[pl.ds(j * chunk + g * LANES, LANES)] = rating

    pltpu.sync_copy(outv, out_hbm.at[pl.ds(base, per_w)])


def _forward(uid, iid, user_table, item_table):
    b = uid.shape[0]
    d = user_table.shape[1]
    nw = 32
    per_w = b // nw
    chunk = 64
    num_chunks = per_w // chunk

    mesh = plsc.VectorSubcoreMesh(core_axis_name="c", subcore_axis_name="s")
    kfn = pl.kernel(
        functools.partial(_sc_kernel_body, num_chunks, chunk, d),
        mesh=mesh,
        compiler_params=pltpu.CompilerParams(needs_layout_passes=False),
        out_type=jax.ShapeDtypeStruct((b,), jnp.float32),
        scratch_types=[
            pltpu.VMEM((per_w,), jnp.int32),
            pltpu.VMEM((per_w,), jnp.int32),
            pltpu.VMEM((2, chunk, d), jnp.float32),
            pltpu.VMEM((2, chunk, d), jnp.float32),
            pltpu.VMEM((chunk, LANES), jnp.float32),
            pltpu.VMEM((per_w,), jnp.float32),
            pltpu.SemaphoreType.DMA,
            pltpu.SemaphoreType.DMA,
            pltpu.SemaphoreType.DMA,
            pltpu.SemaphoreType.DMA,
        ],
    )
    return kfn(uid, iid, user_table, item_table)


def kernel(inputs, user_table, item_table):
    ids = inputs.astype(jnp.int32)
    return _forward(ids[:, 0], ids[:, 1], user_table, item_table)


# speedup vs baseline: 1.0666x; 1.0666x over previous
import functools

import jax
import jax.numpy as jnp
from jax import lax
from jax.experimental import pallas as pl
from jax.experimental.pallas import tpu as pltpu
from jax.experimental.pallas import tpu_sc as plsc

LANES = 16


def _sc_kernel_body(num_chunks, chunk, d, uid_hbm, iid_hbm, ut_hbm, it_hbm,
                    out_hbm, idx_u, idx_i, ubuf, ibuf, dots, outv,
                    sem_u0, sem_u1, sem_i0, sem_i1):
    nc = 2
    wid = lax.axis_index("s") * nc + lax.axis_index("c")
    per_w = num_chunks * chunk
    base = wid * per_w
    d_steps = d // LANES

    pltpu.sync_copy(uid_hbm.at[pl.ds(base, per_w)], idx_u)
    pltpu.sync_copy(iid_hbm.at[pl.ds(base, per_w)], idx_i)

    def issue(j, slot, su, si):
        pltpu.async_copy(ut_hbm.at[idx_u.at[pl.ds(j * chunk, chunk)]],
                         ubuf.at[slot], su)
        pltpu.async_copy(it_hbm.at[idx_i.at[pl.ds(j * chunk, chunk)]],
                         ibuf.at[slot], si)

    def wait(slot, su, si):
        pltpu.make_async_copy(ut_hbm.at[idx_u.at[pl.ds(0, chunk)]],
                              ubuf.at[slot], su).wait()
        pltpu.make_async_copy(it_hbm.at[idx_i.at[pl.ds(0, chunk)]],
                              ibuf.at[slot], si).wait()

    issue(0, 0, sem_u0, sem_i0)
    issue(1, 1, sem_u1, sem_i1)

    lane = lax.iota(jnp.int32, LANES)

    def chunk_body(j, carry):
        s = j % 2

        def slot0(_):
            wait(0, sem_u0, sem_i0)
            return 0

        def slot1(_):
            wait(1, sem_u1, sem_i1)
            return 0

        lax.cond(s == 0, slot0, slot1, 0)

        @plsc.parallel_loop(0, chunk, unroll=4)
        def pair_body(p):
            m = [ubuf[s, p, pl.ds(k * LANES, LANES)] *
                 ibuf[s, p, pl.ds(k * LANES, LANES)] for k in range(d_steps)]
            while len(m) > 1:
                m = [m[2 * t] + m[2 * t + 1] for t in range(len(m) // 2)]
            dots[p, pl.ds(0, LANES)] = m[0]

        @plsc.parallel_loop(0, chunk // LANES, unroll=2)
        def group_body(g):
            row = g * LANES + lane
            acc = [plsc.load_gather(dots,
                                    [row, jnp.full((LANES,), c, jnp.int32)])
                   for c in range(LANES)]
            while len(acc) > 1:
                acc = [acc[2 * t] + acc[2 * t + 1]
                       for t in range(len(acc) // 2)]
            rating = 1.0 + 4.0 / (1.0 + jnp.exp(-acc[0]))
            outv[pl.ds(j * chunk + g * LANES, LANES)] = rating

        nxt = jnp.minimum(j + 2, num_chunks - 1)

        def refill0(_):
            issue(nxt, 0, sem_u0, sem_i0)
            return 0

        def refill1(_):
            issue(nxt, 1, sem_u1, sem_i1)
            return 0

        lax.cond(s == 0, refill0, refill1, 0)
        return carry

    lax.fori_loop(0, num_chunks, chunk_body, 0)

    wait(0, sem_u0, sem_i0)
    wait(1, sem_u1, sem_i1)

    pltpu.sync_copy(outv, out_hbm.at[pl.ds(base, per_w)])


def _forward(uid, iid, user_table, item_table):
    b = uid.shape[0]
    d = user_table.shape[1]
    nw = 32
    per_w = b // nw
    chunk = 64
    num_chunks = per_w // chunk

    mesh = plsc.VectorSubcoreMesh(core_axis_name="c", subcore_axis_name="s")
    kfn = pl.kernel(
        functools.partial(_sc_kernel_body, num_chunks, chunk, d),
        mesh=mesh,
        compiler_params=pltpu.CompilerParams(needs_layout_passes=False),
        out_type=jax.ShapeDtypeStruct((b,), jnp.float32),
        scratch_types=[
            pltpu.VMEM((per_w,), jnp.int32),
            pltpu.VMEM((per_w,), jnp.int32),
            pltpu.VMEM((2, chunk, d), jnp.float32),
            pltpu.VMEM((2, chunk, d), jnp.float32),
            pltpu.VMEM((chunk, LANES), jnp.float32),
            pltpu.VMEM((per_w,), jnp.float32),
            pltpu.SemaphoreType.DMA,
            pltpu.SemaphoreType.DMA,
            pltpu.SemaphoreType.DMA,
            pltpu.SemaphoreType.DMA,
        ],
    )
    return kfn(uid, iid, user_table, item_table)


def kernel(inputs, user_table, item_table):
    ids = inputs.astype(jnp.int32)
    return _forward(ids[:, 0], ids[:, 1], user_table, item_table)


# speedup vs baseline: 1.1106x; 1.0412x over previous
import functools

import jax
import jax.numpy as jnp
from jax import lax
from jax.experimental import pallas as pl
from jax.experimental.pallas import tpu as pltpu
from jax.experimental.pallas import tpu_sc as plsc

LANES = 16


def _sc_kernel_body(num_chunks, chunk, d, uid_hbm, iid_hbm, ut_hbm, it_hbm,
                    out_hbm, idx_u, idx_i, ubuf, ibuf, dots, outv,
                    sem_u0, sem_u1, sem_i0, sem_i1):
    nc = 2
    wid = lax.axis_index("s") * nc + lax.axis_index("c")
    per_w = num_chunks * chunk
    base = wid * per_w
    d_steps = d // LANES

    pltpu.sync_copy(uid_hbm.at[pl.ds(base, per_w)], idx_u)
    pltpu.sync_copy(iid_hbm.at[pl.ds(base, per_w)], idx_i)

    def issue(j, slot, su, si):
        pltpu.async_copy(ut_hbm.at[idx_u.at[pl.ds(j * chunk, chunk)]],
                         ubuf.at[slot], su)
        pltpu.async_copy(it_hbm.at[idx_i.at[pl.ds(j * chunk, chunk)]],
                         ibuf.at[slot], si)

    def wait(slot, su, si):
        pltpu.make_async_copy(ut_hbm.at[idx_u.at[pl.ds(0, chunk)]],
                              ubuf.at[slot], su).wait()
        pltpu.make_async_copy(it_hbm.at[idx_i.at[pl.ds(0, chunk)]],
                              ibuf.at[slot], si).wait()

    issue(0, 0, sem_u0, sem_i0)
    issue(1, 1, sem_u1, sem_i1)

    lane = lax.iota(jnp.int32, LANES)

    def chunk_body(j, carry):
        s = j % 2

        def slot0(_):
            wait(0, sem_u0, sem_i0)
            return 0

        def slot1(_):
            wait(1, sem_u1, sem_i1)
            return 0

        lax.cond(s == 0, slot0, slot1, 0)

        @plsc.parallel_loop(0, chunk, unroll=4)
        def pair_body(p):
            m = [ubuf[s, p, pl.ds(k * LANES, LANES)] *
                 ibuf[s, p, pl.ds(k * LANES, LANES)] for k in range(d_steps)]
            while len(m) > 1:
                m = [m[2 * t] + m[2 * t + 1] for t in range(len(m) // 2)]
            dots[p, pl.ds(0, LANES)] = m[0]

        @plsc.parallel_loop(0, chunk // LANES, unroll=2)
        def group_body(g):
            row = g * LANES + lane
            acc = [plsc.load_gather(dots,
                                    [row, jnp.full((LANES,), c, jnp.int32)])
                   for c in range(LANES)]
            while len(acc) > 1:
                acc = [acc[2 * t] + acc[2 * t + 1]
                       for t in range(len(acc) // 2)]
            rating = 1.0 + 4.0 / (1.0 + jnp.exp(-acc[0]))
            outv[pl.ds(j * chunk + g * LANES, LANES)] = rating

        nxt = j + 2

        def refill0(_):
            issue(nxt, 0, sem_u0, sem_i0)
            return 0

        def refill1(_):
            issue(nxt, 1, sem_u1, sem_i1)
            return 0

        def norefill(_):
            return 0

        branch = jnp.where(nxt < num_chunks, s, 2)
        lax.switch(branch, [refill0, refill1, norefill], 0)
        return carry

    lax.fori_loop(0, num_chunks, chunk_body, 0)

    pltpu.sync_copy(outv, out_hbm.at[pl.ds(base, per_w)])


def _forward(uid, iid, user_table, item_table):
    b = uid.shape[0]
    d = user_table.shape[1]
    nw = 32
    per_w = b // nw
    chunk = 64
    num_chunks = per_w // chunk

    mesh = plsc.VectorSubcoreMesh(core_axis_name="c", subcore_axis_name="s")
    kfn = pl.kernel(
        functools.partial(_sc_kernel_body, num_chunks, chunk, d),
        mesh=mesh,
        compiler_params=pltpu.CompilerParams(needs_layout_passes=False),
        out_type=jax.ShapeDtypeStruct((b,), jnp.float32),
        scratch_types=[
            pltpu.VMEM((per_w,), jnp.int32),
            pltpu.VMEM((per_w,), jnp.int32),
            pltpu.VMEM((2, chunk, d), jnp.float32),
            pltpu.VMEM((2, chunk, d), jnp.float32),
            pltpu.VMEM((chunk, LANES), jnp.float32),
            pltpu.VMEM((per_w,), jnp.float32),
            pltpu.SemaphoreType.DMA,
            pltpu.SemaphoreType.DMA,
            pltpu.SemaphoreType.DMA,
            pltpu.SemaphoreType.DMA,
        ],
    )
    return kfn(uid, iid, user_table, item_table)


def kernel(inputs, user_table, item_table):
    ids = inputs.astype(jnp.int32)
    return _forward(ids[:, 0], ids[:, 1], user_table, item_table)


# speedup vs baseline: 1.1286x; 1.0163x over previous
import functools

import jax
import jax.numpy as jnp
from jax import lax
from jax.experimental import pallas as pl
from jax.experimental.pallas import tpu as pltpu
from jax.experimental.pallas import tpu_sc as plsc

LANES = 16


def _sc_kernel_body(num_chunks, chunk, d, uid_hbm, iid_hbm, ut_hbm, it_hbm,
                    out_hbm, idx_u, idx_i, ubuf, ibuf, dots, outv,
                    sem_u0, sem_u1, sem_i0, sem_i1):
    nc = 2
    wid = lax.axis_index("s") * nc + lax.axis_index("c")
    per_w = num_chunks * chunk
    base = wid * per_w
    d_steps = d // LANES

    cu = pltpu.async_copy(uid_hbm.at[pl.ds(base, per_w)], idx_u, sem_u0)
    ci = pltpu.async_copy(iid_hbm.at[pl.ds(base, per_w)], idx_i, sem_i0)
    cu.wait()
    ci.wait()

    def issue(j, slot, su, si):
        pltpu.async_copy(ut_hbm.at[idx_u.at[pl.ds(j * chunk, chunk)]],
                         ubuf.at[slot], su)
        pltpu.async_copy(it_hbm.at[idx_i.at[pl.ds(j * chunk, chunk)]],
                         ibuf.at[slot], si)

    def wait(slot, su, si):
        pltpu.make_async_copy(ut_hbm.at[idx_u.at[pl.ds(0, chunk)]],
                              ubuf.at[slot], su).wait()
        pltpu.make_async_copy(it_hbm.at[idx_i.at[pl.ds(0, chunk)]],
                              ibuf.at[slot], si).wait()

    issue(0, 0, sem_u0, sem_i0)
    issue(1, 1, sem_u1, sem_i1)

    lane = lax.iota(jnp.int32, LANES)

    def chunk_body(j, carry):
        s = j % 2

        def slot0(_):
            wait(0, sem_u0, sem_i0)
            return 0

        def slot1(_):
            wait(1, sem_u1, sem_i1)
            return 0

        lax.cond(s == 0, slot0, slot1, 0)

        @plsc.parallel_loop(0, chunk, unroll=4)
        def pair_body(p):
            m = [ubuf[s, p, pl.ds(k * LANES, LANES)] *
                 ibuf[s, p, pl.ds(k * LANES, LANES)] for k in range(d_steps)]
            while len(m) > 1:
                m = [m[2 * t] + m[2 * t + 1] for t in range(len(m) // 2)]
            dots[p, pl.ds(0, LANES)] = m[0]

        @plsc.parallel_loop(0, chunk // LANES, unroll=2)
        def group_body(g):
            row = g * LANES + lane
            acc = [plsc.load_gather(dots,
                                    [row, jnp.full((LANES,), c, jnp.int32)])
                   for c in range(LANES)]
            while len(acc) > 1:
                acc = [acc[2 * t] + acc[2 * t + 1]
                       for t in range(len(acc) // 2)]
            rating = 1.0 + 4.0 / (1.0 + jnp.exp(-acc[0]))
            outv[pl.ds(j * chunk + g * LANES, LANES)] = rating

        nxt = j + 2

        def refill0(_):
            issue(nxt, 0, sem_u0, sem_i0)
            return 0

        def refill1(_):
            issue(nxt, 1, sem_u1, sem_i1)
            return 0

        def norefill(_):
            return 0

        branch = jnp.where(nxt < num_chunks, s, 2)
        lax.switch(branch, [refill0, refill1, norefill], 0)
        return carry

    lax.fori_loop(0, num_chunks, chunk_body, 0)

    pltpu.sync_copy(outv, out_hbm.at[pl.ds(base, per_w)])


def _forward(uid, iid, user_table, item_table):
    b = uid.shape[0]
    d = user_table.shape[1]
    nw = 32
    per_w = b // nw
    chunk = 64
    num_chunks = per_w // chunk

    mesh = plsc.VectorSubcoreMesh(core_axis_name="c", subcore_axis_name="s")
    kfn = pl.kernel(
        functools.partial(_sc_kernel_body, num_chunks, chunk, d),
        mesh=mesh,
        compiler_params=pltpu.CompilerParams(needs_layout_passes=False),
        out_type=jax.ShapeDtypeStruct((b,), jnp.float32),
        scratch_types=[
            pltpu.VMEM((per_w,), jnp.int32),
            pltpu.VMEM((per_w,), jnp.int32),
            pltpu.VMEM((2, chunk, d), jnp.float32),
            pltpu.VMEM((2, chunk, d), jnp.float32),
            pltpu.VMEM((chunk, LANES), jnp.float32),
            pltpu.VMEM((per_w,), jnp.float32),
            pltpu.SemaphoreType.DMA,
            pltpu.SemaphoreType.DMA,
            pltpu.SemaphoreType.DMA,
            pltpu.SemaphoreType.DMA,
        ],
    )
    return kfn(uid, iid, user_table, item_table)


def kernel(inputs, user_table, item_table):
    ids = inputs.astype(jnp.int32)
    return _forward(ids[:, 0], ids[:, 1], user_table, item_table)
